# baseline (speedup 1.0000x reference)
import functools

import jax
import jax.numpy as jnp
from jax import lax
from jax.experimental import pallas as pl
from jax.experimental.pallas import tpu as pltpu
from jax.experimental.pallas import tpu_sc as plsc

_PIECE = 16384
_NB = 6
_LOOK = 3
_CB = 16384


class _Ring:
    def __init__(self, buf, sins, souts, src_slice, dst_slice, n):
        self.buf, self.sins, self.souts = buf, sins, souts
        self.src, self.dst, self.n = src_slice, dst_slice, n
        self.gh = [None] * _NB
        self.sh = [None] * _NB

    def _gather(self, k):
        b = k % _NB
        if self.sh[b] is not None:
            self.sh[b].wait()
            self.sh[b] = None
        self.gh[b] = pltpu.async_copy(self.src(k), self.buf.at[b], self.sins[b])

    def prime(self):
        for k in range(min(_LOOK, self.n)):
            self._gather(k)

    def step(self, k):
        b = k % _NB
        self.gh[b].wait()
        self.sh[b] = pltpu.async_copy(self.buf.at[b], self.dst(k), self.souts[b])
        if k + _LOOK < self.n:
            self._gather(k + _LOOK)

    def drain(self):
        for b in range(_NB):
            if self.sh[b] is not None:
                self.sh[b].wait()


def _sc_permute_indices(values, W, C):
    info = plsc.get_sparse_core_info()
    nc, ns = info.num_cores, info.num_subcores
    nw = nc * ns
    per_w = (W * W) // nw
    npieces = C // _PIECE
    n = per_w * npieces

    mesh = plsc.VectorSubcoreMesh(core_axis_name="c", subcore_axis_name="s")

    @functools.partial(
        pl.kernel,
        mesh=mesh,
        out_type=jax.ShapeDtypeStruct((W, W * C), jnp.int32),
        scratch_types=[pltpu.VMEM((_NB, _PIECE), jnp.int32)]
        + [pltpu.SemaphoreType.DMA] * (2 * _NB),
    )
    def k(vals_hbm, out_hbm, vbuf, *sems):
        sin, sout = sems[:_NB], sems[_NB:]
        wid = lax.axis_index("s") * nc + lax.axis_index("c")
        coords = []
        for t in range(per_w):
            p = wid * per_w + t
            coords.append((p // W, p % W))

        def src(k_):
            s, w = coords[k_ // npieces]
            return vals_hbm.at[s, w, pl.ds((k_ % npieces) * _PIECE, _PIECE)]

        def dst(k_):
            s, w = coords[k_ // npieces]
            return out_hbm.at[w, pl.ds(s * C + (k_ % npieces) * _PIECE, _PIECE)]

        ring = _Ring(vbuf, sin, sout, src, dst, n)
        ring.prime()
        for kk in range(n):
            ring.step(kk)
        ring.drain()

    return k(values)


def _tc_weights_and_constants(weights, W, C, N):
    ncb = N // _CB
    nblk = ncb + 1

    def body(w_hbm, out_w, len_ref, off_ref, sem):
        j = pl.program_id(0)

        @pl.when(j == 0)
        def _start():
            for s in range(W):
                for w in range(W):
                    pltpu.make_async_copy(
                        w_hbm.at[s, w], out_w.at[w, pl.ds(s * C, C)], sem
                    ).start()

        len_ref[...] = jnp.ones((W, _CB), jnp.int32)
        off_ref[...] = j * _CB + lax.broadcasted_iota(jnp.int32, (W, _CB), 1)

        @pl.when(j == nblk - 1)
        def _wait():
            for s in range(W):
                for w in range(W):
                    pltpu.make_async_copy(
                        w_hbm.at[s, w], out_w.at[w, pl.ds(s * C, C)], sem
                    ).wait()

    return pl.pallas_call(
        body,
        grid=(nblk,),
        in_specs=[pl.BlockSpec(memory_space=pltpu.MemorySpace.HBM)],
        out_specs=[
            pl.BlockSpec(memory_space=pltpu.MemorySpace.HBM),
            pl.BlockSpec((W, _CB), lambda j: (0, jnp.minimum(j, ncb - 1))),
            pl.BlockSpec((W, _CB), lambda j: (0, j)),
        ],
        out_shape=[
            jax.ShapeDtypeStruct((W, W * C), jnp.float32),
            jax.ShapeDtypeStruct((W, N), jnp.int32),
            jax.ShapeDtypeStruct((W, N + 1), jnp.int32),
        ],
        scratch_shapes=[pltpu.SemaphoreType.DMA],
    )(weights)


def kernel(values, weights):
    W, _, C = values.shape
    N = W * C
    out_indices = _sc_permute_indices(values, W, C)
    out_weights, kjt_lengths, kjt_offsets = _tc_weights_and_constants(
        weights, W, C, N
    )
    return out_indices, out_weights, kjt_lengths, kjt_offsets


# speedup vs baseline: 3.2336x; 3.2336x over previous
import functools

import jax
import jax.numpy as jnp
from jax import lax
from jax.experimental import pallas as pl
from jax.experimental.pallas import tpu as pltpu
from jax.experimental.pallas import tpu_sc as plsc

_PIECE = 16384
_NB = 6
_LOOK = 3
_CB = 16384


class _Ring:
    def __init__(self, buf, sins, souts, src_slice, dst_slice, n):
        self.buf, self.sins, self.souts = buf, sins, souts
        self.src, self.dst, self.n = src_slice, dst_slice, n
        self.gh = [None] * _NB
        self.sh = [None] * _NB

    def _gather(self, k):
        b = k % _NB
        if self.sh[b] is not None:
            self.sh[b].wait()
            self.sh[b] = None
        self.gh[b] = pltpu.async_copy(self.src(k), self.buf.at[b], self.sins[b])

    def prime(self):
        for k in range(min(_LOOK, self.n)):
            self._gather(k)

    def step(self, k):
        b = k % _NB
        self.gh[b].wait()
        self.sh[b] = pltpu.async_copy(self.buf.at[b], self.dst(k), self.souts[b])
        if k + _LOOK < self.n:
            self._gather(k + _LOOK)

    def drain(self):
        for b in range(_NB):
            if self.sh[b] is not None:
                self.sh[b].wait()


def _sc_permute_indices(values, W, C):
    info = plsc.get_sparse_core_info()
    nc, ns = info.num_cores, info.num_subcores
    nw = nc * ns
    per_w = (W * W) // nw
    npieces = C // _PIECE
    n = per_w * npieces

    mesh = plsc.VectorSubcoreMesh(core_axis_name="c", subcore_axis_name="s")

    @functools.partial(
        pl.kernel,
        mesh=mesh,
        out_type=jax.ShapeDtypeStruct((W, W * C), jnp.int32),
        scratch_types=[pltpu.VMEM((_NB, _PIECE), jnp.int32)]
        + [pltpu.SemaphoreType.DMA] * (2 * _NB),
    )
    def k(vals_hbm, out_hbm, vbuf, *sems):
        sin, sout = sems[:_NB], sems[_NB:]
        wid = lax.axis_index("s") * nc + lax.axis_index("c")
        coords = []
        for t in range(per_w):
            p = wid * per_w + t
            coords.append((p // W, p % W))

        def src(k_):
            s, w = coords[k_ // npieces]
            return vals_hbm.at[s, w, pl.ds((k_ % npieces) * _PIECE, _PIECE)]

        def dst(k_):
            s, w = coords[k_ // npieces]
            return out_hbm.at[w, pl.ds(s * C + (k_ % npieces) * _PIECE, _PIECE)]

        ring = _Ring(vbuf, sin, sout, src, dst, n)
        ring.prime()
        for kk in range(n):
            ring.step(kk)
        ring.drain()

    return k(values)


def _tc_weights_and_constants(weights, W, C, N):
    nsteps = W * W
    lcb = N // nsteps
    ocb = (N + nsteps) // nsteps
    ocb = ((ocb + 127) // 128) * 128

    def body(w_ref, out_w_ref, len_ref, off_ref):
        p = pl.program_id(0)
        out_w_ref[...] = w_ref[...]
        len_ref[...] = jnp.ones((W, lcb), jnp.int32)
        off_ref[...] = p * ocb + lax.broadcasted_iota(jnp.int32, (W, ocb), 1)


    out_w, lengths, offsets = pl.pallas_call(
        body,
        grid=(nsteps,),
        in_specs=[pl.BlockSpec((1, 1, C), lambda p: (p, 0, 0))],
        out_specs=[
            pl.BlockSpec((1, 1, C), lambda p: ((p % W) * W + p // W, 0, 0)),
            pl.BlockSpec((W, lcb), lambda p: (0, p)),
            pl.BlockSpec((W, ocb), lambda p: (0, p)),
        ],
        out_shape=[
            jax.ShapeDtypeStruct((W * W, 1, C), jnp.float32),
            jax.ShapeDtypeStruct((W, N), jnp.int32),
            jax.ShapeDtypeStruct((W, N + 1), jnp.int32),
        ],
    )(weights.reshape(W * W, 1, C))
    return out_w.reshape(W, W * C), lengths, offsets


def kernel(values, weights):
    W, _, C = values.shape
    N = W * C
    out_indices = _sc_permute_indices(values, W, C)
    out_weights, kjt_lengths, kjt_offsets = _tc_weights_and_constants(
        weights, W, C, N
    )
    return out_indices, out_weights, kjt_lengths, kjt_offsets


# speedup vs baseline: 12.6603x; 3.9152x over previous
import functools

import jax
import jax.numpy as jnp
from jax import lax
from jax.experimental import pallas as pl
from jax.experimental.pallas import tpu as pltpu
from jax.experimental.pallas import tpu_sc as plsc

_PIECE = 16384
_NB = 6
_LOOK = 3
_CB = 16384


class _Ring:
    def __init__(self, buf, sins, souts, src_slice, dst_slice, n):
        self.buf, self.sins, self.souts = buf, sins, souts
        self.src, self.dst, self.n = src_slice, dst_slice, n
        self.gh = [None] * _NB
        self.sh = [None] * _NB

    def _gather(self, k):
        b = k % _NB
        if self.sh[b] is not None:
            self.sh[b].wait()
            self.sh[b] = None
        self.gh[b] = pltpu.async_copy(self.src(k), self.buf.at[b], self.sins[b])

    def prime(self):
        for k in range(min(_LOOK, self.n)):
            self._gather(k)

    def step(self, k):
        b = k % _NB
        self.gh[b].wait()
        self.sh[b] = pltpu.async_copy(self.buf.at[b], self.dst(k), self.souts[b])
        if k + _LOOK < self.n:
            self._gather(k + _LOOK)

    def drain(self):
        for b in range(_NB):
            if self.sh[b] is not None:
                self.sh[b].wait()


def _sc_permute_indices(values, W, C):
    info = plsc.get_sparse_core_info()
    nc, ns = info.num_cores, info.num_subcores
    nw = nc * ns
    per_w = (W * W) // nw
    npieces = C // _PIECE
    n = per_w * npieces

    mesh = plsc.VectorSubcoreMesh(core_axis_name="c", subcore_axis_name="s")

    @functools.partial(
        pl.kernel,
        mesh=mesh,
        out_type=jax.ShapeDtypeStruct((W, W * C), jnp.int32),
        scratch_types=[pltpu.VMEM((_NB, _PIECE), jnp.int32)]
        + [pltpu.SemaphoreType.DMA] * (2 * _NB),
    )
    def k(vals_hbm, out_hbm, vbuf, *sems):
        sin, sout = sems[:_NB], sems[_NB:]
        wid = lax.axis_index("s") * nc + lax.axis_index("c")
        coords = []
        for t in range(per_w):
            p = wid * per_w + t
            coords.append((p // W, p % W))

        def src(k_):
            s, w = coords[k_ // npieces]
            return vals_hbm.at[s, w, pl.ds((k_ % npieces) * _PIECE, _PIECE)]

        def dst(k_):
            s, w = coords[k_ // npieces]
            return out_hbm.at[w, pl.ds(s * C + (k_ % npieces) * _PIECE, _PIECE)]

        ring = _Ring(vbuf, sin, sout, src, dst, n)
        ring.prime()
        for kk in range(n):
            ring.step(kk)
        ring.drain()

    return k(values)


def _tc_weights_and_constants(weights, W, C, N):
    ncol = 2
    cb = C // ncol
    nsteps = W * ncol
    lcb = N // nsteps
    ocb = ((N + nsteps) // nsteps + 127) // 128 * 128

    def body(w_ref, out_w_ref, len_ref, off_ref):
        i = pl.program_id(0)
        j = pl.program_id(1)
        p = i * ncol + j
        out_w_ref[...] = w_ref[0]
        len_ref[...] = jnp.ones((W, lcb), jnp.int32)
        off_ref[...] = p * ocb + lax.broadcasted_iota(jnp.int32, (W, ocb), 1)

    return pl.pallas_call(
        body,
        grid=(W, ncol),
        in_specs=[pl.BlockSpec((1, W, cb), lambda i, j: (i, 0, j))],
        out_specs=[
            pl.BlockSpec((W, cb), lambda i, j: (0, i * ncol + j)),
            pl.BlockSpec((W, lcb), lambda i, j: (0, i * ncol + j)),
            pl.BlockSpec((W, ocb), lambda i, j: (0, i * ncol + j)),
        ],
        out_shape=[
            jax.ShapeDtypeStruct((W, W * C), jnp.float32),
            jax.ShapeDtypeStruct((W, N), jnp.int32),
            jax.ShapeDtypeStruct((W, N + 1), jnp.int32),
        ],
    )(weights)


def kernel(values, weights):
    W, _, C = values.shape
    N = W * C
    out_indices = _sc_permute_indices(values, W, C)
    out_weights, kjt_lengths, kjt_offsets = _tc_weights_and_constants(
        weights, W, C, N
    )
    return out_indices, out_weights, kjt_lengths, kjt_offsets
